# Initial kernel scaffold; baseline (speedup 1.0000x reference)
import jax
import jax.numpy as jnp
from jax.experimental import pallas as pl


def kernel(vertices, triangle_indices, mvp, resolution):
    raise NotImplementedError("write your pallas kernel here")


# speedup vs baseline: 4.1974x; 4.1974x over previous
import functools

import jax
import jax.numpy as jnp
from jax import lax
from jax.experimental import pallas as pl
from jax.experimental.pallas import tpu as pltpu
from jax.experimental.pallas import tpu_sc as plsc

V = 8192
NUM_TRIS = 4096
RES = 256
TRI_CHUNK = 128
NUM_CHUNKS = NUM_TRIS // TRI_CHUNK
STRIP = 32
HALF = 128


def _sc_gather(table16, idx):
    info = plsc.get_sparse_core_info()
    nc, ns = info.num_cores, info.num_subcores
    nw = nc * ns
    b = idx.shape[0]
    b_per_w = b // nw
    mesh = plsc.VectorSubcoreMesh(core_axis_name="c", subcore_axis_name="s")

    @functools.partial(
        pl.kernel,
        out_type=jax.ShapeDtypeStruct((b, 16), jnp.float32),
        mesh=mesh,
        scratch_types=[
            pltpu.VMEM((b_per_w,), jnp.int32),
            pltpu.VMEM((b_per_w, 16), jnp.float32),
            pltpu.SemaphoreType.DMA,
        ],
        compiler_params=pltpu.CompilerParams(use_tc_tiling_on_sc=False),
    )
    def gather_kernel(table_hbm, idx_hbm, out_hbm, idx_v, rows_v, sem):
        wid = lax.axis_index("s") * nc + lax.axis_index("c")
        base = wid * b_per_w
        pltpu.sync_copy(idx_hbm.at[pl.ds(base, b_per_w)], idx_v)
        pltpu.async_copy(table_hbm.at[idx_v], rows_v, sem).wait()
        pltpu.sync_copy(rows_v, out_hbm.at[pl.ds(base, b_per_w)])

    return gather_kernel(table16, idx)


def _raster_body(g_ref, b0_ref, b1_ref, b2_ref, tf_ref, zbuf):
    pi = pl.program_id(0)

    @pl.when(pi == 0)
    def _init():
        zbuf[...] = jnp.full((RES, RES), 1.0e10, jnp.float32)
        zero = jnp.zeros((RES, RES), jnp.float32)
        b0_ref[...] = zero
        b1_ref[...] = zero
        b2_ref[...] = zero
        tf_ref[...] = zero

    px = lax.broadcasted_iota(jnp.int32, (STRIP, HALF), 1)
    py = lax.broadcasted_iota(jnp.int32, (STRIP, HALF), 0)

    def tri_body(t, carry):
        r = 3 * t
        x0 = g_ref[r, 0].astype(jnp.int32)
        y0 = g_ref[r, 1].astype(jnp.int32)
        z0 = g_ref[r, 2]
        x1 = g_ref[r + 1, 0].astype(jnp.int32)
        y1 = g_ref[r + 1, 1].astype(jnp.int32)
        z1 = g_ref[r + 1, 2]
        x2 = g_ref[r + 2, 0].astype(jnp.int32)
        y2 = g_ref[r + 2, 1].astype(jnp.int32)
        z2 = g_ref[r + 2, 2]
        a0 = y1 - y2
        b0 = x2 - x1
        a1 = y2 - y0
        b1 = x0 - x2
        c0 = -(a0 * x2 + b0 * y2)
        c1 = -(a1 * x2 + b1 * y2)
        dint = a0 * x0 + b0 * y0 + c0
        df = dint.astype(jnp.float32)
        rb = 1.0 / df
        xmin = jnp.minimum(jnp.minimum(x0, x1), x2)
        xmax = jnp.maximum(jnp.maximum(x0, x1), x2)
        ymin = jnp.minimum(jnp.minimum(y0, y1), y2)
        ymax = jnp.maximum(jnp.maximum(y0, y1), y2)
        ok = ((dint != 0) & (xmax >= 0) & (xmin < RES)
              & (ymax >= 0) & (ymin < RES))

        @pl.when(ok)
        def _tri():
            p0 = a0 * px + b0 * py
            p1 = a1 * px + b1 * py
            s_lo = jnp.maximum(ymin, 0) // STRIP
            s_hi = jnp.minimum(ymax, RES - 1) // STRIP
            h_lo = jnp.maximum(xmin, 0) // HALF
            h_hi = jnp.minimum(xmax, RES - 1) // HALF
            tid1 = (pi * TRI_CHUNK + t + 1).astype(jnp.float32)

            def strip_body(s, c2):
                row = pl.multiple_of(s * STRIP, STRIP)
                for h in (0, 1):
                    @pl.when((h_lo <= h) & (h <= h_hi))
                    def _half(h=h):
                        off0 = c0 + b0 * (s * STRIP) + a0 * (HALF * h)
                        off1 = c1 + b1 * (s * STRIP) + a1 * (HALF * h)
                        w0 = (p0 + off0).astype(jnp.float32)
                        w1 = (p1 + off1).astype(jnp.float32)
                        bw0 = w0 * rb
                        bw1 = w1 * rb
                        bw2 = (1.0 - bw0) - bw1
                        z = bw0 * z0 + bw1 * z1 + bw2 * z2
                        valid = jnp.minimum(jnp.minimum(bw0, bw1), bw2) >= 0.0
                        zm = jnp.where(valid, z, jnp.inf)
                        rows = pl.ds(row, STRIP)
                        cols = pl.ds(h * HALF, HALF)
                        zb = zbuf[rows, cols]
                        upd = zm < zb
                        zbuf[rows, cols] = jnp.where(upd, zm, zb)
                        b0_ref[rows, cols] = jnp.where(upd, bw0, b0_ref[rows, cols])
                        b1_ref[rows, cols] = jnp.where(upd, bw1, b1_ref[rows, cols])
                        b2_ref[rows, cols] = jnp.where(upd, bw2, b2_ref[rows, cols])
                        tf_ref[rows, cols] = jnp.where(upd, tid1, tf_ref[rows, cols])
                return c2

            lax.fori_loop(s_lo, s_hi + 1, strip_body, 0)

        return carry

    lax.fori_loop(0, TRI_CHUNK, tri_body, 0)


def _rasterize_tc(g3, interpret=False):
    out_shape = [jax.ShapeDtypeStruct((RES, RES), jnp.float32)] * 4
    out_spec = pl.BlockSpec((RES, RES), lambda i: (0, 0))
    return pl.pallas_call(
        _raster_body,
        grid=(NUM_CHUNKS,),
        in_specs=[pl.BlockSpec((3 * TRI_CHUNK, 3), lambda i: (i, 0),
                               memory_space=pltpu.SMEM)],
        out_specs=[out_spec] * 4,
        out_shape=out_shape,
        scratch_shapes=[pltpu.VMEM((RES, RES), jnp.float32)],
        interpret=interpret,
    )(g3)


def _vertex_screen(vertices, mvp, resolution):
    res_f = jnp.asarray(resolution, jnp.float32)
    num_v = vertices.shape[0]
    mvp_m = mvp.reshape(4, 4).astype(jnp.float32)
    homo = jnp.concatenate(
        [vertices.astype(jnp.float32), jnp.ones((num_v, 1), jnp.float32)], axis=1)
    clip = homo @ mvp_m.T
    w = clip[:, 3]
    safe_w = jnp.where(w != 0.0, w, 1.0)
    ndc = jnp.where((w != 0.0)[:, None], clip[:, :3] / safe_w[:, None], clip[:, :3])
    fx = jnp.nan_to_num((ndc[:, 0] + 1.0) * 0.5 * res_f, nan=0.0,
                        posinf=3.0e4, neginf=-3.0e4)
    fy = jnp.nan_to_num((ndc[:, 1] + 1.0) * 0.5 * res_f, nan=0.0,
                        posinf=3.0e4, neginf=-3.0e4)
    xs = jnp.clip(fx, -12000.0, 12000.0).astype(jnp.int32)
    ys = jnp.clip(fy, -12000.0, 12000.0).astype(jnp.int32)
    zs = ndc[:, 2]
    return xs, ys, zs


def kernel(vertices, triangle_indices, mvp, resolution):
    xs, ys, zs = _vertex_screen(vertices, mvp, resolution)
    table = jnp.zeros((V, 16), jnp.float32)
    table = table.at[:, 0].set(xs.astype(jnp.float32))
    table = table.at[:, 1].set(ys.astype(jnp.float32))
    table = table.at[:, 2].set(zs)
    gathered = _sc_gather(table, triangle_indices.astype(jnp.int32))
    g3 = gathered[:, :3]
    b0, b1, b2, tf = _rasterize_tc(g3)
    return jnp.stack([b0, b1, b2, tf], axis=-1)


# speedup vs baseline: 4.6446x; 1.1066x over previous
import functools

import jax
import jax.numpy as jnp
from jax import lax
from jax.experimental import pallas as pl
from jax.experimental.pallas import tpu as pltpu
from jax.experimental.pallas import tpu_sc as plsc

V = 8192
NUM_TRIS = 4096
RES = 256
TRI_CHUNK = 128
NUM_CHUNKS = NUM_TRIS // TRI_CHUNK
STRIP = 32
HALF = 128


def _sc_gather(table16, idx):
    info = plsc.get_sparse_core_info()
    nc, ns = info.num_cores, info.num_subcores
    nw = nc * ns
    b = idx.shape[0]
    b_per_w = b // nw
    mesh = plsc.VectorSubcoreMesh(core_axis_name="c", subcore_axis_name="s")

    @functools.partial(
        pl.kernel,
        out_type=jax.ShapeDtypeStruct((b, 16), jnp.float32),
        mesh=mesh,
        scratch_types=[
            pltpu.VMEM((b_per_w,), jnp.int32),
            pltpu.VMEM((b_per_w, 16), jnp.float32),
            pltpu.SemaphoreType.DMA,
        ],
        compiler_params=pltpu.CompilerParams(use_tc_tiling_on_sc=False),
    )
    def gather_kernel(table_hbm, idx_hbm, out_hbm, idx_v, rows_v, sem):
        wid = lax.axis_index("s") * nc + lax.axis_index("c")
        base = wid * b_per_w
        pltpu.sync_copy(idx_hbm.at[pl.ds(base, b_per_w)], idx_v)
        pltpu.async_copy(table_hbm.at[idx_v], rows_v, sem).wait()
        pltpu.sync_copy(rows_v, out_hbm.at[pl.ds(base, b_per_w)])

    return gather_kernel(table16, idx)


NCORES = 1
CROWS = RES // NCORES


NSTRIPS = CROWS // STRIP


def _raster_body(g_ref, b0_ref, b1_ref, b2_ref, tf_ref, zbuf, hiz):
    ci = pl.program_id(0)
    pi = pl.program_id(1)
    rbase = ci * CROWS

    @pl.when(pi == 0)
    def _init():
        zbuf[...] = jnp.full((CROWS, RES), 1.0e10, jnp.float32)
        zero = jnp.zeros((CROWS, RES), jnp.float32)
        b0_ref[...] = zero
        b1_ref[...] = zero
        b2_ref[...] = zero
        tf_ref[...] = zero
        for s in range(NSTRIPS):
            for h in range(2):
                hiz[s, h] = 1.0e10

    @pl.when(pi > 0)
    def _refresh_hiz():
        for s in range(NSTRIPS):
            for h in range(2):
                hiz[s, h] = jnp.max(zbuf[pl.ds(s * STRIP, STRIP),
                                         pl.ds(h * HALF, HALF)])

    px = lax.broadcasted_iota(jnp.int32, (STRIP, HALF), 1)
    py = lax.broadcasted_iota(jnp.int32, (STRIP, HALF), 0)

    def tri_body(t, carry):
        r = 3 * t
        x0 = g_ref[r, 0].astype(jnp.int32)
        y0 = g_ref[r, 1].astype(jnp.int32)
        z0 = g_ref[r, 2]
        x1 = g_ref[r + 1, 0].astype(jnp.int32)
        y1 = g_ref[r + 1, 1].astype(jnp.int32)
        z1 = g_ref[r + 1, 2]
        x2 = g_ref[r + 2, 0].astype(jnp.int32)
        y2 = g_ref[r + 2, 1].astype(jnp.int32)
        z2 = g_ref[r + 2, 2]
        a0 = y1 - y2
        b0 = x2 - x1
        a1 = y2 - y0
        b1 = x0 - x2
        c0 = -(a0 * x2 + b0 * y2)
        c1 = -(a1 * x2 + b1 * y2)
        dint = a0 * x0 + b0 * y0 + c0
        df = dint.astype(jnp.float32)
        rb = 1.0 / df
        xmin = jnp.minimum(jnp.minimum(x0, x1), x2)
        xmax = jnp.maximum(jnp.maximum(x0, x1), x2)
        ymin = jnp.minimum(jnp.minimum(y0, y1), y2)
        ymax = jnp.maximum(jnp.maximum(y0, y1), y2)
        ok = ((dint != 0) & (xmax >= 0) & (xmin < RES)
              & (ymax >= rbase) & (ymin < rbase + CROWS))
        zmin3 = jnp.minimum(jnp.minimum(z0, z1), z2)
        zam = jnp.maximum(jnp.maximum(jnp.abs(z0), jnp.abs(z1)), jnp.abs(z2))
        zlow = zmin3 - 1.0e-5 * zam

        @pl.when(ok)
        def _tri():
            p0 = a0 * px + b0 * py
            p1 = a1 * px + b1 * py
            s_lo = (jnp.maximum(ymin - rbase, 0)) // STRIP
            s_hi = (jnp.minimum(ymax - rbase, CROWS - 1)) // STRIP
            h_lo = jnp.maximum(xmin, 0) // HALF
            h_hi = jnp.minimum(xmax, RES - 1) // HALF
            tid1 = (pi * TRI_CHUNK + t + 1).astype(jnp.float32)

            def strip_body(s, c2):
                row = pl.multiple_of(s * STRIP, STRIP)
                for h in (0, 1):
                    @pl.when((h_lo <= h) & (h <= h_hi) & (zlow <= hiz[s, h]))
                    def _half(h=h):
                        yb = rbase + s * STRIP
                        off0 = c0 + b0 * yb + a0 * (HALF * h)
                        off1 = c1 + b1 * yb + a1 * (HALF * h)
                        w0 = (p0 + off0).astype(jnp.float32)
                        w1 = (p1 + off1).astype(jnp.float32)
                        bw0 = w0 * rb
                        bw1 = w1 * rb
                        bw2 = (1.0 - bw0) - bw1
                        z = bw0 * z0 + bw1 * z1 + bw2 * z2
                        valid = jnp.minimum(jnp.minimum(bw0, bw1), bw2) >= 0.0
                        zm = jnp.where(valid, z, jnp.inf)
                        rows = pl.ds(row, STRIP)
                        cols = pl.ds(h * HALF, HALF)
                        zb = zbuf[rows, cols]
                        upd = zm < zb
                        zbuf[rows, cols] = jnp.where(upd, zm, zb)
                        b0_ref[rows, cols] = jnp.where(upd, bw0, b0_ref[rows, cols])
                        b1_ref[rows, cols] = jnp.where(upd, bw1, b1_ref[rows, cols])
                        b2_ref[rows, cols] = jnp.where(upd, bw2, b2_ref[rows, cols])
                        tf_ref[rows, cols] = jnp.where(upd, tid1, tf_ref[rows, cols])
                return c2

            lax.fori_loop(s_lo, s_hi + 1, strip_body, 0)

        return carry

    lax.fori_loop(0, TRI_CHUNK, tri_body, 0)


def _rasterize_tc(g3, interpret=False):
    out_shape = [jax.ShapeDtypeStruct((RES, RES), jnp.float32)] * 4
    out_spec = pl.BlockSpec((CROWS, RES), lambda c, i: (c, 0))
    return pl.pallas_call(
        _raster_body,
        grid=(NCORES, NUM_CHUNKS),
        in_specs=[pl.BlockSpec((3 * TRI_CHUNK, 3), lambda c, i: (i, 0),
                               memory_space=pltpu.SMEM)],
        out_specs=[out_spec] * 4,
        out_shape=out_shape,
        scratch_shapes=[pltpu.VMEM((CROWS, RES), jnp.float32),
                        pltpu.SMEM((NSTRIPS, 2), jnp.float32)],
        compiler_params=pltpu.CompilerParams(
            dimension_semantics=("arbitrary", "arbitrary")),
        interpret=interpret,
    )(g3)


def _vertex_screen(vertices, mvp, resolution):
    res_f = jnp.asarray(resolution, jnp.float32)
    num_v = vertices.shape[0]
    mvp_m = mvp.reshape(4, 4).astype(jnp.float32)
    homo = jnp.concatenate(
        [vertices.astype(jnp.float32), jnp.ones((num_v, 1), jnp.float32)], axis=1)
    clip = homo @ mvp_m.T
    w = clip[:, 3]
    safe_w = jnp.where(w != 0.0, w, 1.0)
    ndc = jnp.where((w != 0.0)[:, None], clip[:, :3] / safe_w[:, None], clip[:, :3])
    fx = jnp.nan_to_num((ndc[:, 0] + 1.0) * 0.5 * res_f, nan=0.0,
                        posinf=3.0e4, neginf=-3.0e4)
    fy = jnp.nan_to_num((ndc[:, 1] + 1.0) * 0.5 * res_f, nan=0.0,
                        posinf=3.0e4, neginf=-3.0e4)
    xs = jnp.clip(fx, -12000.0, 12000.0).astype(jnp.int32)
    ys = jnp.clip(fy, -12000.0, 12000.0).astype(jnp.int32)
    zs = ndc[:, 2]
    return xs, ys, zs


def kernel(vertices, triangle_indices, mvp, resolution):
    xs, ys, zs = _vertex_screen(vertices, mvp, resolution)
    table = jnp.zeros((V, 16), jnp.float32)
    table = table.at[:, 0].set(xs.astype(jnp.float32))
    table = table.at[:, 1].set(ys.astype(jnp.float32))
    table = table.at[:, 2].set(zs)
    gathered = _sc_gather(table, triangle_indices.astype(jnp.int32))
    g3 = gathered[:, :3]
    b0, b1, b2, tf = _rasterize_tc(g3)
    return jnp.stack([b0, b1, b2, tf], axis=-1)
